# baseline (speedup 1.0000x reference)
import functools

import jax
import jax.numpy as jnp
from jax import lax
from jax.experimental import pallas as pl
from jax.experimental.pallas import tpu as pltpu
from jax.experimental.pallas import tpu_sc as plsc

N_NODES = 10000
D = 128
NP = 10240
E = 320000
NTILES = 32
EPT = 10240
EP = NTILES * EPT
CH = 128
NCH = EPT // CH
CHA = 64
NBUF = 3
KG = 2
NCHA0 = 96
NCHA1 = 224
NR0 = NCHA0 // 2
NR1 = NCHA1 // 2
RPT = NP // 16
BN = 2048

_mesh = plsc.VectorSubcoreMesh(core_axis_name="c", subcore_axis_name="s")


def _deg_body(dst_hbm, out_hbm, dst_v, ones_v, buf_v, deg_sh):
    c = lax.axis_index("c")
    s = lax.axis_index("s")
    wid = c * 16 + s
    pltpu.sync_copy(dst_hbm.at[wid], dst_v)
    zeros = jnp.zeros((16,), jnp.float32)
    ones = jnp.ones((16,), jnp.float32)
    for k in range(CH // 16):
        ones_v[pl.ds(k * 16, 16)] = ones
    for k in range(RPT // 16):
        buf_v[pl.ds(k * 16, 16)] = zeros
    pltpu.sync_copy(buf_v, deg_sh.at[pl.ds(s * RPT, RPT)])
    plsc.subcore_barrier()

    def body(j, carry):
        pltpu.sync_copy(ones_v, deg_sh.at[dst_v.at[j]], add=True)
        return carry

    lax.fori_loop(0, NCH, body, 0)
    plsc.subcore_barrier()
    pltpu.sync_copy(deg_sh.at[pl.ds(s * RPT, RPT)], buf_v)
    pltpu.sync_copy(buf_v, out_hbm.at[c, pl.ds(s * RPT, RPT)])


_deg_call = functools.partial(
    pl.kernel,
    out_type=jax.ShapeDtypeStruct((2, NP), jnp.float32),
    mesh=_mesh,
    scratch_types=[
        pltpu.VMEM((NCH, CH), jnp.int32),
        pltpu.VMEM((CH,), jnp.float32),
        pltpu.VMEM((RPT,), jnp.float32),
        pltpu.VMEM_SHARED((NP,), jnp.float32),
    ],
)(_deg_body)


def _prep_body(p0_ref, p1_ref, x_ref, y_ref, z_ref, dis_ref):
    deg = p0_ref[...] + p1_ref[...] + 1.0
    dis = lax.rsqrt(deg)
    x = x_ref[...]
    y_ref[...] = x * dis[:, None]
    z_ref[...] = x * (1.0 / deg)[:, None]
    dis_ref[...] = dis[:, None]


_prep_call = pl.pallas_call(
    _prep_body,
    grid=(NP // BN,),
    in_specs=[
        pl.BlockSpec((BN,), lambda i: (i,)),
        pl.BlockSpec((BN,), lambda i: (i,)),
        pl.BlockSpec((BN, D), lambda i: (i, 0)),
    ],
    out_specs=[
        pl.BlockSpec((BN, D), lambda i: (i, 0)),
        pl.BlockSpec((BN, D), lambda i: (i, 0)),
        pl.BlockSpec((BN, 1), lambda i: (i, 0)),
    ],
    out_shape=[
        jax.ShapeDtypeStruct((NP, D), jnp.float32),
        jax.ShapeDtypeStruct((NP, D), jnp.float32),
        jax.ShapeDtypeStruct((NP, 1), jnp.float32),
    ],
)


def _agg_body(y_hbm, combo_hbm, zrow_hbm, out_hbm,
              combo_v, sidx_v, didx_v, buf_v, acc_sh, semg, sems):
    c = lax.axis_index("c")
    s = lax.axis_index("s")
    ncha = jnp.where(c == 0, NCHA0, NCHA1)

    @pl.when(c == 0)
    def _():
        pltpu.sync_copy(combo_hbm.at[pl.ds(s * NR0, NR0)],
                        combo_v.at[pl.ds(0, NR0)])

    @pl.when(c == 1)
    def _():
        pltpu.sync_copy(combo_hbm.at[pl.ds(16 * NR0 + s * NR1, NR1)], combo_v)

    pltpu.sync_copy(zrow_hbm, buf_v.at[0])
    for k in range(RPT // CHA):
        pltpu.sync_copy(buf_v.at[0], acc_sh.at[pl.ds(s * RPT + k * CHA, CHA)])
    plsc.subcore_barrier()

    cpr = CH // CHA

    def unpack(j, slot):
        r = j // cpr
        c0 = lax.rem(j, cpr) * CHA
        for k in range(CHA // 16):
            v = combo_v[r, pl.ds(c0 + k * 16, 16)]
            sidx_v[slot, pl.ds(k * 16, 16)] = jnp.bitwise_and(v, 0xFFFF)
            didx_v[slot, pl.ds(k * 16, 16)] = lax.shift_right_logical(v, 16)

    def issue_gather(j, slot):
        pltpu.async_copy(y_hbm.at[sidx_v.at[slot]], buf_v.at[slot],
                         semg.at[slot])

    for j in range(KG):
        unpack(j, j)
        issue_gather(j, j)

    def body(j, carry):
        fslot = lax.rem(j + KG, NBUF)
        jslot = lax.rem(j, NBUF)

        @pl.when(j >= NBUF - KG)
        def _():
            pltpu.make_async_copy(buf_v.at[fslot],
                                  acc_sh.at[didx_v.at[fslot]],
                                  sems.at[fslot]).wait()

        @pl.when(j + KG < ncha)
        def _():
            unpack(j + KG, fslot)
            issue_gather(j + KG, fslot)

        pltpu.make_async_copy(y_hbm.at[sidx_v.at[jslot]], buf_v.at[jslot],
                              semg.at[jslot]).wait()
        pltpu.async_copy(buf_v.at[jslot], acc_sh.at[didx_v.at[jslot]],
                         sems.at[jslot], add=True)
        return carry

    lax.fori_loop(0, ncha, body, 0)
    for t in range(NBUF - KG):
        slot = lax.rem(ncha - (NBUF - KG) + t, NBUF)
        pltpu.make_async_copy(buf_v.at[slot], acc_sh.at[didx_v.at[slot]],
                              sems.at[slot]).wait()
    plsc.subcore_barrier()
    for k in range(RPT // CHA):
        r0 = s * RPT + k * CHA
        pltpu.sync_copy(acc_sh.at[pl.ds(r0, CHA)], buf_v.at[0])
        pltpu.sync_copy(buf_v.at[0], out_hbm.at[c, pl.ds(r0, CHA)])


_agg_call = functools.partial(
    pl.kernel,
    out_type=jax.ShapeDtypeStruct((2, NP, D), jnp.float32),
    mesh=_mesh,
    scratch_types=[
        pltpu.VMEM((NR1, CH), jnp.int32),
        pltpu.VMEM((NBUF, CHA), jnp.int32),
        pltpu.VMEM((NBUF, CHA), jnp.int32),
        pltpu.VMEM((NBUF, CHA, D), jnp.float32),
        pltpu.VMEM_SHARED((NP, D), jnp.float32),
        pltpu.SemaphoreType.DMA((NBUF,)),
        pltpu.SemaphoreType.DMA((NBUF,)),
    ],
)(_agg_body)


def _out_body(s0_ref, s1_ref, z_ref, dis_ref, w_ref, b_ref, o_ref):
    agg = dis_ref[...] * (s0_ref[...] + s1_ref[...]) + z_ref[...]
    acc = jnp.dot(agg, w_ref[...], preferred_element_type=jnp.float32,
                  precision=lax.Precision.HIGHEST)
    o_ref[...] = jnp.maximum(acc + b_ref[...], 0.0)


_out_call = pl.pallas_call(
    _out_body,
    grid=(NP // BN,),
    in_specs=[
        pl.BlockSpec((BN, D), lambda i: (i, 0)),
        pl.BlockSpec((BN, D), lambda i: (i, 0)),
        pl.BlockSpec((BN, D), lambda i: (i, 0)),
        pl.BlockSpec((BN, 1), lambda i: (i, 0)),
        pl.BlockSpec((D, D), lambda i: (0, 0)),
        pl.BlockSpec((1, D), lambda i: (0, 0)),
    ],
    out_specs=pl.BlockSpec((BN, D), lambda i: (i, 0)),
    out_shape=jax.ShapeDtypeStruct((NP, D), jnp.float32),
)


def kernel(x, edge_index, W, b):
    ei = edge_index.astype(jnp.int32)
    pad = jnp.full((EP - E,), N_NODES, jnp.int32)
    src_p = jnp.concatenate([ei[0], pad])
    dst_p = jnp.concatenate([ei[1], pad])
    x_p = jnp.concatenate(
        [x.astype(jnp.float32), jnp.zeros((NP - N_NODES, D), jnp.float32)])
    zrow = jnp.zeros((CHA, D), jnp.float32)

    combo = (src_p | (dst_p << 16)).reshape(EP // CH, CH)

    degp = _deg_call(dst_p.reshape(NTILES, NCH, CH))
    y, z, dis = _prep_call(degp[0], degp[1], x_p)
    sp = _agg_call(y, combo, zrow)
    out = _out_call(sp[0], sp[1], z, dis, W.astype(jnp.float32),
                    b.astype(jnp.float32).reshape(1, D))
    return out[:N_NODES]


# speedup vs baseline: 1.0433x; 1.0433x over previous
import functools

import jax
import jax.numpy as jnp
from jax import lax
from jax.experimental import pallas as pl
from jax.experimental.pallas import tpu as pltpu
from jax.experimental.pallas import tpu_sc as plsc

N_NODES = 10000
D = 128
NP = 10240
E = 320000
NTILES = 32
EPT = 10240
EP = NTILES * EPT
CH = 128
NCH = EPT // CH
CHA = 64
NBUF = 3
KG = 2
NCHA0 = 224
NCHA1 = 96
NR0 = NCHA0 // 2
NR1 = NCHA1 // 2
RPT = NP // 16
BN = 2048

_mesh = plsc.VectorSubcoreMesh(core_axis_name="c", subcore_axis_name="s")


def _deg_body(dst_hbm, out_hbm, dst_v, ones_v, buf_v, deg_sh):
    c = lax.axis_index("c")
    s = lax.axis_index("s")
    wid = c * 16 + s
    pltpu.sync_copy(dst_hbm.at[wid], dst_v)
    zeros = jnp.zeros((16,), jnp.float32)
    ones = jnp.ones((16,), jnp.float32)
    for k in range(CH // 16):
        ones_v[pl.ds(k * 16, 16)] = ones
    for k in range(RPT // 16):
        buf_v[pl.ds(k * 16, 16)] = zeros
    pltpu.sync_copy(buf_v, deg_sh.at[pl.ds(s * RPT, RPT)])
    plsc.subcore_barrier()

    def body(j, carry):
        pltpu.sync_copy(ones_v, deg_sh.at[dst_v.at[j]], add=True)
        return carry

    lax.fori_loop(0, NCH, body, 0)
    plsc.subcore_barrier()
    pltpu.sync_copy(deg_sh.at[pl.ds(s * RPT, RPT)], buf_v)
    pltpu.sync_copy(buf_v, out_hbm.at[c, pl.ds(s * RPT, RPT)])


_deg_call = functools.partial(
    pl.kernel,
    out_type=jax.ShapeDtypeStruct((2, NP), jnp.float32),
    mesh=_mesh,
    scratch_types=[
        pltpu.VMEM((NCH, CH), jnp.int32),
        pltpu.VMEM((CH,), jnp.float32),
        pltpu.VMEM((RPT,), jnp.float32),
        pltpu.VMEM_SHARED((NP,), jnp.float32),
    ],
)(_deg_body)


def _prep_body(p0_ref, p1_ref, x_ref, y_ref, z_ref, dis_ref):
    deg = p0_ref[...] + p1_ref[...] + 1.0
    dis = lax.rsqrt(deg)
    x = x_ref[...]
    y_ref[...] = x * dis[:, None]
    z_ref[...] = x * (1.0 / deg)[:, None]
    dis_ref[...] = dis[:, None]


_prep_call = pl.pallas_call(
    _prep_body,
    grid=(NP // BN,),
    in_specs=[
        pl.BlockSpec((BN,), lambda i: (i,)),
        pl.BlockSpec((BN,), lambda i: (i,)),
        pl.BlockSpec((BN, D), lambda i: (i, 0)),
    ],
    out_specs=[
        pl.BlockSpec((BN, D), lambda i: (i, 0)),
        pl.BlockSpec((BN, D), lambda i: (i, 0)),
        pl.BlockSpec((BN, 1), lambda i: (i, 0)),
    ],
    out_shape=[
        jax.ShapeDtypeStruct((NP, D), jnp.float32),
        jax.ShapeDtypeStruct((NP, D), jnp.float32),
        jax.ShapeDtypeStruct((NP, 1), jnp.float32),
    ],
)


def _agg_body(y_hbm, combo_hbm, zrow_hbm, out_hbm,
              combo_v, sidx_v, didx_v, buf_v, acc_sh, semg, sems):
    c = lax.axis_index("c")
    s = lax.axis_index("s")
    ncha = jnp.where(c == 0, NCHA0, NCHA1)

    @pl.when(c == 0)
    def _():
        pltpu.sync_copy(combo_hbm.at[pl.ds(s * NR0, NR0)],
                        combo_v.at[pl.ds(0, NR0)])

    @pl.when(c == 1)
    def _():
        pltpu.sync_copy(combo_hbm.at[pl.ds(16 * NR0 + s * NR1, NR1)],
                        combo_v.at[pl.ds(0, NR1)])

    pltpu.sync_copy(zrow_hbm, buf_v.at[0])
    for k in range(RPT // CHA):
        pltpu.sync_copy(buf_v.at[0], acc_sh.at[pl.ds(s * RPT + k * CHA, CHA)])
    plsc.subcore_barrier()

    cpr = CH // CHA

    def unpack(j, slot):
        r = j // cpr
        c0 = lax.rem(j, cpr) * CHA
        for k in range(CHA // 16):
            v = combo_v[r, pl.ds(c0 + k * 16, 16)]
            sidx_v[slot, pl.ds(k * 16, 16)] = jnp.bitwise_and(v, 0xFFFF)
            didx_v[slot, pl.ds(k * 16, 16)] = lax.shift_right_logical(v, 16)

    def issue_gather(j, slot):
        pltpu.async_copy(y_hbm.at[sidx_v.at[slot]], buf_v.at[slot],
                         semg.at[slot])

    for j in range(KG):
        unpack(j, j)
        issue_gather(j, j)

    def body(j, carry):
        fslot = lax.rem(j + KG, NBUF)
        jslot = lax.rem(j, NBUF)

        @pl.when(j >= NBUF - KG)
        def _():
            pltpu.make_async_copy(buf_v.at[fslot],
                                  acc_sh.at[didx_v.at[fslot]],
                                  sems.at[fslot]).wait()

        @pl.when(j + KG < ncha)
        def _():
            unpack(j + KG, fslot)
            issue_gather(j + KG, fslot)

        pltpu.make_async_copy(y_hbm.at[sidx_v.at[jslot]], buf_v.at[jslot],
                              semg.at[jslot]).wait()
        pltpu.async_copy(buf_v.at[jslot], acc_sh.at[didx_v.at[jslot]],
                         sems.at[jslot], add=True)
        return carry

    lax.fori_loop(0, ncha, body, 0)
    for t in range(NBUF - KG):
        slot = lax.rem(ncha - (NBUF - KG) + t, NBUF)
        pltpu.make_async_copy(buf_v.at[slot], acc_sh.at[didx_v.at[slot]],
                              sems.at[slot]).wait()
    plsc.subcore_barrier()
    for k in range(RPT // CHA):
        r0 = s * RPT + k * CHA
        pltpu.sync_copy(acc_sh.at[pl.ds(r0, CHA)], buf_v.at[0])
        pltpu.sync_copy(buf_v.at[0], out_hbm.at[c, pl.ds(r0, CHA)])


_agg_call = functools.partial(
    pl.kernel,
    out_type=jax.ShapeDtypeStruct((2, NP, D), jnp.float32),
    mesh=_mesh,
    scratch_types=[
        pltpu.VMEM((max(NR0, NR1), CH), jnp.int32),
        pltpu.VMEM((NBUF, CHA), jnp.int32),
        pltpu.VMEM((NBUF, CHA), jnp.int32),
        pltpu.VMEM((NBUF, CHA, D), jnp.float32),
        pltpu.VMEM_SHARED((NP, D), jnp.float32),
        pltpu.SemaphoreType.DMA((NBUF,)),
        pltpu.SemaphoreType.DMA((NBUF,)),
    ],
)(_agg_body)


def _out_body(s0_ref, s1_ref, z_ref, dis_ref, w_ref, b_ref, o_ref):
    agg = dis_ref[...] * (s0_ref[...] + s1_ref[...]) + z_ref[...]
    acc = jnp.dot(agg, w_ref[...], preferred_element_type=jnp.float32,
                  precision=lax.Precision.HIGHEST)
    o_ref[...] = jnp.maximum(acc + b_ref[...], 0.0)


_out_call = pl.pallas_call(
    _out_body,
    grid=(NP // BN,),
    in_specs=[
        pl.BlockSpec((BN, D), lambda i: (i, 0)),
        pl.BlockSpec((BN, D), lambda i: (i, 0)),
        pl.BlockSpec((BN, D), lambda i: (i, 0)),
        pl.BlockSpec((BN, 1), lambda i: (i, 0)),
        pl.BlockSpec((D, D), lambda i: (0, 0)),
        pl.BlockSpec((1, D), lambda i: (0, 0)),
    ],
    out_specs=pl.BlockSpec((BN, D), lambda i: (i, 0)),
    out_shape=jax.ShapeDtypeStruct((NP, D), jnp.float32),
)


def kernel(x, edge_index, W, b):
    ei = edge_index.astype(jnp.int32)
    pad = jnp.full((EP - E,), N_NODES, jnp.int32)
    src_p = jnp.concatenate([ei[0], pad])
    dst_p = jnp.concatenate([ei[1], pad])
    x_p = jnp.concatenate(
        [x.astype(jnp.float32), jnp.zeros((NP - N_NODES, D), jnp.float32)])
    zrow = jnp.zeros((CHA, D), jnp.float32)

    combo = (src_p | (dst_p << 16)).reshape(EP // CH, CH)

    degp = _deg_call(dst_p.reshape(NTILES, NCH, CH))
    y, z, dis = _prep_call(degp[0], degp[1], x_p)
    sp = _agg_call(y, combo, zrow)
    out = _out_call(sp[0], sp[1], z, dis, W.astype(jnp.float32),
                    b.astype(jnp.float32).reshape(1, D))
    return out[:N_NODES]


# speedup vs baseline: 1.0456x; 1.0022x over previous
import functools

import jax
import jax.numpy as jnp
from jax import lax
from jax.experimental import pallas as pl
from jax.experimental.pallas import tpu as pltpu
from jax.experimental.pallas import tpu_sc as plsc

N_NODES = 10000
D = 128
NP = 10240
E = 320000
NTILES = 32
EPT = 10240
EP = NTILES * EPT
CH = 128
NCH = EPT // CH
CHA = 64
NBUF = 3
KG = 2
NCHA0 = 256
NCHA1 = 64
NR0 = NCHA0 // 2
NR1 = NCHA1 // 2
RPT = NP // 16
BN = 2048

_mesh = plsc.VectorSubcoreMesh(core_axis_name="c", subcore_axis_name="s")


def _deg_body(dst_hbm, out_hbm, dst_v, ones_v, buf_v, deg_sh):
    c = lax.axis_index("c")
    s = lax.axis_index("s")
    wid = c * 16 + s
    pltpu.sync_copy(dst_hbm.at[wid], dst_v)
    zeros = jnp.zeros((16,), jnp.float32)
    ones = jnp.ones((16,), jnp.float32)
    for k in range(CH // 16):
        ones_v[pl.ds(k * 16, 16)] = ones
    for k in range(RPT // 16):
        buf_v[pl.ds(k * 16, 16)] = zeros
    pltpu.sync_copy(buf_v, deg_sh.at[pl.ds(s * RPT, RPT)])
    plsc.subcore_barrier()

    def body(j, carry):
        pltpu.sync_copy(ones_v, deg_sh.at[dst_v.at[j]], add=True)
        return carry

    lax.fori_loop(0, NCH, body, 0)
    plsc.subcore_barrier()
    pltpu.sync_copy(deg_sh.at[pl.ds(s * RPT, RPT)], buf_v)
    pltpu.sync_copy(buf_v, out_hbm.at[c, pl.ds(s * RPT, RPT)])


_deg_call = functools.partial(
    pl.kernel,
    out_type=jax.ShapeDtypeStruct((2, NP), jnp.float32),
    mesh=_mesh,
    scratch_types=[
        pltpu.VMEM((NCH, CH), jnp.int32),
        pltpu.VMEM((CH,), jnp.float32),
        pltpu.VMEM((RPT,), jnp.float32),
        pltpu.VMEM_SHARED((NP,), jnp.float32),
    ],
)(_deg_body)


def _prep_body(p0_ref, p1_ref, x_ref, y_ref, z_ref, dis_ref):
    deg = p0_ref[...] + p1_ref[...] + 1.0
    dis = lax.rsqrt(deg)
    x = x_ref[...]
    y_ref[...] = x * dis[:, None]
    z_ref[...] = x * (1.0 / deg)[:, None]
    dis_ref[...] = dis[:, None]


_prep_call = pl.pallas_call(
    _prep_body,
    grid=(NP // BN,),
    in_specs=[
        pl.BlockSpec((BN,), lambda i: (i,)),
        pl.BlockSpec((BN,), lambda i: (i,)),
        pl.BlockSpec((BN, D), lambda i: (i, 0)),
    ],
    out_specs=[
        pl.BlockSpec((BN, D), lambda i: (i, 0)),
        pl.BlockSpec((BN, D), lambda i: (i, 0)),
        pl.BlockSpec((BN, 1), lambda i: (i, 0)),
    ],
    out_shape=[
        jax.ShapeDtypeStruct((NP, D), jnp.float32),
        jax.ShapeDtypeStruct((NP, D), jnp.float32),
        jax.ShapeDtypeStruct((NP, 1), jnp.float32),
    ],
)


def _agg_body(y_hbm, combo_hbm, zrow_hbm, out_hbm,
              combo_v, sidx_v, didx_v, buf_v, acc_sh, semg, sems):
    c = lax.axis_index("c")
    s = lax.axis_index("s")
    ncha = jnp.where(c == 0, NCHA0, NCHA1)

    @pl.when(c == 0)
    def _():
        pltpu.sync_copy(combo_hbm.at[pl.ds(s * NR0, NR0)],
                        combo_v.at[pl.ds(0, NR0)])

    @pl.when(c == 1)
    def _():
        pltpu.sync_copy(combo_hbm.at[pl.ds(16 * NR0 + s * NR1, NR1)],
                        combo_v.at[pl.ds(0, NR1)])

    pltpu.sync_copy(zrow_hbm, buf_v.at[0])
    for k in range(RPT // CHA):
        pltpu.sync_copy(buf_v.at[0], acc_sh.at[pl.ds(s * RPT + k * CHA, CHA)])
    plsc.subcore_barrier()

    cpr = CH // CHA

    def unpack(j, slot):
        r = j // cpr
        c0 = lax.rem(j, cpr) * CHA
        for k in range(CHA // 16):
            v = combo_v[r, pl.ds(c0 + k * 16, 16)]
            sidx_v[slot, pl.ds(k * 16, 16)] = jnp.bitwise_and(v, 0xFFFF)
            didx_v[slot, pl.ds(k * 16, 16)] = lax.shift_right_logical(v, 16)

    def issue_gather(j, slot):
        pltpu.async_copy(y_hbm.at[sidx_v.at[slot]], buf_v.at[slot],
                         semg.at[slot])

    for j in range(KG):
        unpack(j, j)
        issue_gather(j, j)

    def body(j, carry):
        fslot = lax.rem(j + KG, NBUF)
        jslot = lax.rem(j, NBUF)

        @pl.when(j >= NBUF - KG)
        def _():
            pltpu.make_async_copy(buf_v.at[fslot],
                                  acc_sh.at[didx_v.at[fslot]],
                                  sems.at[fslot]).wait()

        @pl.when(j + KG < ncha)
        def _():
            unpack(j + KG, fslot)
            issue_gather(j + KG, fslot)

        pltpu.make_async_copy(y_hbm.at[sidx_v.at[jslot]], buf_v.at[jslot],
                              semg.at[jslot]).wait()
        pltpu.async_copy(buf_v.at[jslot], acc_sh.at[didx_v.at[jslot]],
                         sems.at[jslot], add=True)
        return carry

    lax.fori_loop(0, ncha, body, 0)
    for t in range(NBUF - KG):
        slot = lax.rem(ncha - (NBUF - KG) + t, NBUF)
        pltpu.make_async_copy(buf_v.at[slot], acc_sh.at[didx_v.at[slot]],
                              sems.at[slot]).wait()
    plsc.subcore_barrier()
    for k in range(RPT // CHA):
        r0 = s * RPT + k * CHA
        pltpu.sync_copy(acc_sh.at[pl.ds(r0, CHA)], buf_v.at[0])
        pltpu.sync_copy(buf_v.at[0], out_hbm.at[c, pl.ds(r0, CHA)])


_agg_call = functools.partial(
    pl.kernel,
    out_type=jax.ShapeDtypeStruct((2, NP, D), jnp.float32),
    mesh=_mesh,
    scratch_types=[
        pltpu.VMEM((max(NR0, NR1), CH), jnp.int32),
        pltpu.VMEM((NBUF, CHA), jnp.int32),
        pltpu.VMEM((NBUF, CHA), jnp.int32),
        pltpu.VMEM((NBUF, CHA, D), jnp.float32),
        pltpu.VMEM_SHARED((NP, D), jnp.float32),
        pltpu.SemaphoreType.DMA((NBUF,)),
        pltpu.SemaphoreType.DMA((NBUF,)),
    ],
)(_agg_body)


def _out_body(s0_ref, s1_ref, z_ref, dis_ref, w_ref, b_ref, o_ref):
    agg = dis_ref[...] * (s0_ref[...] + s1_ref[...]) + z_ref[...]
    acc = jnp.dot(agg, w_ref[...], preferred_element_type=jnp.float32,
                  precision=lax.Precision.HIGHEST)
    o_ref[...] = jnp.maximum(acc + b_ref[...], 0.0)


_out_call = pl.pallas_call(
    _out_body,
    grid=(NP // BN,),
    in_specs=[
        pl.BlockSpec((BN, D), lambda i: (i, 0)),
        pl.BlockSpec((BN, D), lambda i: (i, 0)),
        pl.BlockSpec((BN, D), lambda i: (i, 0)),
        pl.BlockSpec((BN, 1), lambda i: (i, 0)),
        pl.BlockSpec((D, D), lambda i: (0, 0)),
        pl.BlockSpec((1, D), lambda i: (0, 0)),
    ],
    out_specs=pl.BlockSpec((BN, D), lambda i: (i, 0)),
    out_shape=jax.ShapeDtypeStruct((NP, D), jnp.float32),
)


def kernel(x, edge_index, W, b):
    ei = edge_index.astype(jnp.int32)
    pad = jnp.full((EP - E,), N_NODES, jnp.int32)
    src_p = jnp.concatenate([ei[0], pad])
    dst_p = jnp.concatenate([ei[1], pad])
    x_p = jnp.concatenate(
        [x.astype(jnp.float32), jnp.zeros((NP - N_NODES, D), jnp.float32)])
    zrow = jnp.zeros((CHA, D), jnp.float32)

    combo = (src_p | (dst_p << 16)).reshape(EP // CH, CH)

    degp = _deg_call(dst_p.reshape(NTILES, NCH, CH))
    y, z, dis = _prep_call(degp[0], degp[1], x_p)
    sp = _agg_call(y, combo, zrow)
    out = _out_call(sp[0], sp[1], z, dis, W.astype(jnp.float32),
                    b.astype(jnp.float32).reshape(1, D))
    return out[:N_NODES]


# speedup vs baseline: 1.0886x; 1.0411x over previous
import functools

import jax
import jax.numpy as jnp
from jax import lax
from jax.experimental import pallas as pl
from jax.experimental.pallas import tpu as pltpu
from jax.experimental.pallas import tpu_sc as plsc

N_NODES = 10000
D = 128
NP = 10240
E = 320000
NTILES = 32
EPT = 10240
EP = NTILES * EPT
CH = 128
NCH = EPT // CH
CHA = 64
NBUF = 3
KG = 2
NCHA0 = 288
NCHA1 = 32
NR0 = NCHA0 // 2
NR1 = NCHA1 // 2
RPT = NP // 16
BN = 2048

_mesh = plsc.VectorSubcoreMesh(core_axis_name="c", subcore_axis_name="s")


def _deg_body(dst_hbm, out_hbm, dst_v, ones_v, buf_v, deg_sh):
    c = lax.axis_index("c")
    s = lax.axis_index("s")
    wid = c * 16 + s
    pltpu.sync_copy(dst_hbm.at[wid], dst_v)
    zeros = jnp.zeros((16,), jnp.float32)
    ones = jnp.ones((16,), jnp.float32)
    for k in range(CH // 16):
        ones_v[pl.ds(k * 16, 16)] = ones
    for k in range(RPT // 16):
        buf_v[pl.ds(k * 16, 16)] = zeros
    pltpu.sync_copy(buf_v, deg_sh.at[pl.ds(s * RPT, RPT)])
    plsc.subcore_barrier()

    def body(j, carry):
        pltpu.sync_copy(ones_v, deg_sh.at[dst_v.at[j]], add=True)
        return carry

    lax.fori_loop(0, NCH, body, 0)
    plsc.subcore_barrier()
    pltpu.sync_copy(deg_sh.at[pl.ds(s * RPT, RPT)], buf_v)
    pltpu.sync_copy(buf_v, out_hbm.at[c, pl.ds(s * RPT, RPT)])


_deg_call = functools.partial(
    pl.kernel,
    out_type=jax.ShapeDtypeStruct((2, NP), jnp.float32),
    mesh=_mesh,
    scratch_types=[
        pltpu.VMEM((NCH, CH), jnp.int32),
        pltpu.VMEM((CH,), jnp.float32),
        pltpu.VMEM((RPT,), jnp.float32),
        pltpu.VMEM_SHARED((NP,), jnp.float32),
    ],
)(_deg_body)


def _prep_body(p0_ref, p1_ref, x_ref, y_ref, z_ref, dis_ref):
    deg = p0_ref[...] + p1_ref[...] + 1.0
    dis = lax.rsqrt(deg)
    x = x_ref[...]
    y_ref[...] = x * dis[:, None]
    z_ref[...] = x * (1.0 / deg)[:, None]
    dis_ref[...] = dis[:, None]


_prep_call = pl.pallas_call(
    _prep_body,
    grid=(NP // BN,),
    in_specs=[
        pl.BlockSpec((BN,), lambda i: (i,)),
        pl.BlockSpec((BN,), lambda i: (i,)),
        pl.BlockSpec((BN, D), lambda i: (i, 0)),
    ],
    out_specs=[
        pl.BlockSpec((BN, D), lambda i: (i, 0)),
        pl.BlockSpec((BN, D), lambda i: (i, 0)),
        pl.BlockSpec((BN, 1), lambda i: (i, 0)),
    ],
    out_shape=[
        jax.ShapeDtypeStruct((NP, D), jnp.float32),
        jax.ShapeDtypeStruct((NP, D), jnp.float32),
        jax.ShapeDtypeStruct((NP, 1), jnp.float32),
    ],
)


def _agg_body(y_hbm, combo_hbm, zrow_hbm, out_hbm,
              combo_v, sidx_v, didx_v, buf_v, acc_sh, semg, sems):
    c = lax.axis_index("c")
    s = lax.axis_index("s")
    ncha = jnp.where(c == 0, NCHA0, NCHA1)

    @pl.when(c == 0)
    def _():
        pltpu.sync_copy(combo_hbm.at[pl.ds(s * NR0, NR0)],
                        combo_v.at[pl.ds(0, NR0)])

    @pl.when(c == 1)
    def _():
        pltpu.sync_copy(combo_hbm.at[pl.ds(16 * NR0 + s * NR1, NR1)],
                        combo_v.at[pl.ds(0, NR1)])

    pltpu.sync_copy(zrow_hbm, buf_v.at[0])
    for k in range(RPT // CHA):
        pltpu.sync_copy(buf_v.at[0], acc_sh.at[pl.ds(s * RPT + k * CHA, CHA)])
    plsc.subcore_barrier()

    cpr = CH // CHA

    def unpack(j, slot):
        r = j // cpr
        c0 = lax.rem(j, cpr) * CHA
        for k in range(CHA // 16):
            v = combo_v[r, pl.ds(c0 + k * 16, 16)]
            sidx_v[slot, pl.ds(k * 16, 16)] = jnp.bitwise_and(v, 0xFFFF)
            didx_v[slot, pl.ds(k * 16, 16)] = lax.shift_right_logical(v, 16)

    def issue_gather(j, slot):
        pltpu.async_copy(y_hbm.at[sidx_v.at[slot]], buf_v.at[slot],
                         semg.at[slot])

    for j in range(KG):
        unpack(j, j)
        issue_gather(j, j)

    def body(j, carry):
        fslot = lax.rem(j + KG, NBUF)
        jslot = lax.rem(j, NBUF)

        @pl.when(j >= NBUF - KG)
        def _():
            pltpu.make_async_copy(buf_v.at[fslot],
                                  acc_sh.at[didx_v.at[fslot]],
                                  sems.at[fslot]).wait()

        @pl.when(j + KG < ncha)
        def _():
            unpack(j + KG, fslot)
            issue_gather(j + KG, fslot)

        pltpu.make_async_copy(y_hbm.at[sidx_v.at[jslot]], buf_v.at[jslot],
                              semg.at[jslot]).wait()
        pltpu.async_copy(buf_v.at[jslot], acc_sh.at[didx_v.at[jslot]],
                         sems.at[jslot], add=True)
        return carry

    lax.fori_loop(0, ncha, body, 0)
    for t in range(NBUF - KG):
        slot = lax.rem(ncha - (NBUF - KG) + t, NBUF)
        pltpu.make_async_copy(buf_v.at[slot], acc_sh.at[didx_v.at[slot]],
                              sems.at[slot]).wait()
    plsc.subcore_barrier()
    for k in range(RPT // CHA):
        r0 = s * RPT + k * CHA
        pltpu.sync_copy(acc_sh.at[pl.ds(r0, CHA)], buf_v.at[0])
        pltpu.sync_copy(buf_v.at[0], out_hbm.at[c, pl.ds(r0, CHA)])


_agg_call = functools.partial(
    pl.kernel,
    out_type=jax.ShapeDtypeStruct((2, NP, D), jnp.float32),
    mesh=_mesh,
    scratch_types=[
        pltpu.VMEM((max(NR0, NR1), CH), jnp.int32),
        pltpu.VMEM((NBUF, CHA), jnp.int32),
        pltpu.VMEM((NBUF, CHA), jnp.int32),
        pltpu.VMEM((NBUF, CHA, D), jnp.float32),
        pltpu.VMEM_SHARED((NP, D), jnp.float32),
        pltpu.SemaphoreType.DMA((NBUF,)),
        pltpu.SemaphoreType.DMA((NBUF,)),
    ],
)(_agg_body)


def _out_body(s0_ref, s1_ref, z_ref, dis_ref, w_ref, b_ref, o_ref):
    agg = dis_ref[...] * (s0_ref[...] + s1_ref[...]) + z_ref[...]
    acc = jnp.dot(agg, w_ref[...], preferred_element_type=jnp.float32,
                  precision=lax.Precision.HIGHEST)
    o_ref[...] = jnp.maximum(acc + b_ref[...], 0.0)


_out_call = pl.pallas_call(
    _out_body,
    grid=(NP // BN,),
    in_specs=[
        pl.BlockSpec((BN, D), lambda i: (i, 0)),
        pl.BlockSpec((BN, D), lambda i: (i, 0)),
        pl.BlockSpec((BN, D), lambda i: (i, 0)),
        pl.BlockSpec((BN, 1), lambda i: (i, 0)),
        pl.BlockSpec((D, D), lambda i: (0, 0)),
        pl.BlockSpec((1, D), lambda i: (0, 0)),
    ],
    out_specs=pl.BlockSpec((BN, D), lambda i: (i, 0)),
    out_shape=jax.ShapeDtypeStruct((NP, D), jnp.float32),
)


def kernel(x, edge_index, W, b):
    ei = edge_index.astype(jnp.int32)
    pad = jnp.full((EP - E,), N_NODES, jnp.int32)
    src_p = jnp.concatenate([ei[0], pad])
    dst_p = jnp.concatenate([ei[1], pad])
    x_p = jnp.concatenate(
        [x.astype(jnp.float32), jnp.zeros((NP - N_NODES, D), jnp.float32)])
    zrow = jnp.zeros((CHA, D), jnp.float32)

    combo = (src_p | (dst_p << 16)).reshape(EP // CH, CH)

    degp = _deg_call(dst_p.reshape(NTILES, NCH, CH))
    y, z, dis = _prep_call(degp[0], degp[1], x_p)
    sp = _agg_call(y, combo, zrow)
    out = _out_call(sp[0], sp[1], z, dis, W.astype(jnp.float32),
                    b.astype(jnp.float32).reshape(1, D))
    return out[:N_NODES]
